# baseline (speedup 1.0000x reference)
import dataclasses
import functools

import numpy as np
import jax
import jax.numpy as jnp
from jax import lax
from jax.experimental import pallas as pl
from jax.experimental.pallas import tpu as pltpu
from jax.experimental.pallas import tpu_sc as plsc

_HASH_SEED = 1
_DEP = 5
_SLOTS = 5120
_PRIME = 2147483647
_BATCH = 4096
_ROWS = _DEP * _BATCH
_BLK = _SLOTS // 16
_NBLK = _ROWS * _BLK

_NW = 32
_RPW = _ROWS // _NW
_ZROWS = 8
_NZ = _RPW // _ZROWS
_NS = _RPW // 128


def _hash_tables():
    rng = np.random.RandomState(_HASH_SEED)
    A = rng.randint(1, _PRIME, size=(_DEP,)).astype(np.int64)
    B = rng.randint(0, _PRIME, size=(_DEP,)).astype(np.int64)
    v = np.arange(1024, dtype=np.int64)
    T0 = (A[:, None] * v[None, :] + B[:, None]) % _PRIME
    T1 = (A[:, None] * 1024 * v[None, :]) % _PRIME
    return (T0.reshape(-1).astype(np.int32), T1.reshape(-1).astype(np.int32))


_T0, _T1 = _hash_tables()


def _sc_body(x_hbm, t0_hbm, t1_hbm, val_hbm, o_hbm,
             zbuf, x_v, t0_v, t1_v, val_v, srcb, blkb, zsem, insem):
    wid = (lax.axis_index("s").astype(jnp.int32) * jnp.int32(2)
           + lax.axis_index("c").astype(jnp.int32))
    base = wid * jnp.int32(_RPW)
    bbase = base * jnp.int32(_BLK)

    pltpu.make_async_copy(x_hbm, x_v, insem).start()
    pltpu.make_async_copy(t0_hbm, t0_v, insem).start()
    pltpu.make_async_copy(t1_hbm, t1_v, insem).start()
    pltpu.make_async_copy(val_hbm, val_v, insem).start()

    def zrow(i, _):
        zbuf[i, :] = jnp.zeros((16,), jnp.float32)
        return jnp.int32(0)
    lax.fori_loop(jnp.int32(0), jnp.int32(_ZROWS * _BLK), zrow, jnp.int32(0))

    def fire(i, _):
        pltpu.make_async_copy(
            zbuf,
            o_hbm.at[pl.ds(bbase + i * jnp.int32(_ZROWS * _BLK), _ZROWS * _BLK)],
            zsem,
        ).start()
        return jnp.int32(0)
    lax.fori_loop(jnp.int32(0), jnp.int32(_NZ), fire, jnp.int32(0))

    pltpu.make_async_copy(x_hbm, x_v, insem).wait()
    pltpu.make_async_copy(t0_hbm, t0_v, insem).wait()
    pltpu.make_async_copy(t1_hbm, t1_v, insem).wait()
    pltpu.make_async_copy(val_hbm, val_v, insem).wait()

    def zsrow(i, _):
        g = i // jnp.int32(128)
        p = lax.rem(i, jnp.int32(128))
        srcb[g, p, :] = jnp.zeros((16,), jnp.float32)
        return jnp.int32(0)
    lax.fori_loop(jnp.int32(0), jnp.int32(_NS * 128), zsrow, jnp.int32(0))

    lanes = lax.broadcasted_iota(jnp.int32, (16,), 0)

    def hash_vec(v, _):
        g = v // jnp.int32(8)
        c = lax.rem(v, jnp.int32(8))
        r = base + v * jnp.int32(16) + lanes
        d = r >> jnp.int32(12)
        b = r & jnp.int32(4095)
        k = b * jnp.int32(_DEP) + d
        f = k >> jnp.int32(12)
        j = k & jnp.int32(4095)
        xg = plsc.load_gather(x_v, [j])
        x1 = xg >> jnp.int32(10)
        x0 = xg & jnp.int32(1023)
        t1 = plsc.load_gather(t1_v, [f * jnp.int32(1024) + x1])
        t0 = plsc.load_gather(t0_v, [f * jnp.int32(1024) + x0])
        s = t1.astype(jnp.uint32) + t0.astype(jnp.uint32)
        pr = jnp.uint32(_PRIME)
        rm = jnp.where(s >= pr, s - pr, s).astype(jnp.int32)
        slot = lax.rem(rm, jnp.int32(_SLOTS))
        vals = plsc.load_gather(val_v, [k])
        blk = r * jnp.int32(_BLK) + (slot >> jnp.int32(4))
        lane = slot & jnp.int32(15)
        pos = c * jnp.int32(16) + lanes
        gv = jnp.full((16,), jnp.int32(0), jnp.int32) + g
        plsc.store_scatter(srcb, [gv, pos, lane], vals)
        blkb[g, pl.ds(c * jnp.int32(16), 16)] = blk
        return jnp.int32(0)

    lax.fori_loop(jnp.int32(0), jnp.int32(_RPW // 16), hash_vec, jnp.int32(0))

    def drain(i, _):
        pltpu.make_async_copy(
            zbuf,
            o_hbm.at[pl.ds(bbase + i * jnp.int32(_ZROWS * _BLK), _ZROWS * _BLK)],
            zsem,
        ).wait()
        return jnp.int32(0)
    lax.fori_loop(jnp.int32(0), jnp.int32(_NZ), drain, jnp.int32(0))

    for g in range(_NS):
        gi = jnp.int32(g)
        pltpu.sync_copy(srcb.at[gi], o_hbm.at[blkb.at[gi]])


def kernel(input_tensor, values):
    x = input_tensor.astype(jnp.int32)
    t0 = jnp.asarray(_T0)
    t1 = jnp.asarray(_T1)
    vals = values.astype(jnp.float32)

    mesh = plsc.VectorSubcoreMesh(core_axis_name="c", subcore_axis_name="s")
    cp = pltpu.CompilerParams()
    if "needs_layout_passes" in pltpu.CompilerParams.__dataclass_fields__:
        cp = dataclasses.replace(cp, needs_layout_passes=False)
    cp = dataclasses.replace(cp, use_tc_tiling_on_sc=False)
    run = pl.kernel(
        _sc_body,
        compiler_params=cp,
        out_type=jax.ShapeDtypeStruct((_NBLK, 16), jnp.float32),
        mesh=mesh,
        scratch_types=[
            pltpu.VMEM((_ZROWS * _BLK, 16), jnp.float32),
            pltpu.VMEM((_BATCH,), jnp.int32),
            pltpu.VMEM((_DEP * 1024,), jnp.int32),
            pltpu.VMEM((_DEP * 1024,), jnp.int32),
            pltpu.VMEM((_ROWS,), jnp.float32),
            pltpu.VMEM((_NS, 128, 16), jnp.float32),
            pltpu.VMEM((_NS, 128), jnp.int32),
            pltpu.SemaphoreType.DMA,
            pltpu.SemaphoreType.DMA,
        ],
    )
    out = run(x, t0, t1, vals)
    return out.reshape(_DEP, _BATCH, _SLOTS)


# speedup vs baseline: 1.0010x; 1.0010x over previous
import dataclasses
import functools

import numpy as np
import jax
import jax.numpy as jnp
from jax import lax
from jax.experimental import pallas as pl
from jax.experimental.pallas import tpu as pltpu
from jax.experimental.pallas import tpu_sc as plsc

_HASH_SEED = 1
_DEP = 5
_SLOTS = 5120
_PRIME = 2147483647
_BATCH = 4096
_ROWS = _DEP * _BATCH
_BLK = _SLOTS // 128
_NBLK = _ROWS * _BLK

_NW = 32
_RPW = _ROWS // _NW
_ZROWS = 16
_NZ = _RPW // _ZROWS
_NS = _RPW // 128


def _hash_tables():
    rng = np.random.RandomState(_HASH_SEED)
    A = rng.randint(1, _PRIME, size=(_DEP,)).astype(np.int64)
    B = rng.randint(0, _PRIME, size=(_DEP,)).astype(np.int64)
    v = np.arange(1024, dtype=np.int64)
    T0 = (A[:, None] * v[None, :] + B[:, None]) % _PRIME
    T1 = (A[:, None] * 1024 * v[None, :]) % _PRIME
    return (T0.reshape(-1).astype(np.int32), T1.reshape(-1).astype(np.int32))


_T0, _T1 = _hash_tables()


def _sc_body(x_hbm, t0_hbm, t1_hbm, val_hbm, o_hbm,
             zbuf, x_v, t0_v, t1_v, val_v, srcb, blkb, laneb, zsem, insem):
    wid = (lax.axis_index("s").astype(jnp.int32) * jnp.int32(2)
           + lax.axis_index("c").astype(jnp.int32))
    base = wid * jnp.int32(_RPW)
    bbase = base * jnp.int32(_BLK)

    pltpu.make_async_copy(x_hbm, x_v, insem).start()
    pltpu.make_async_copy(t0_hbm, t0_v, insem).start()
    pltpu.make_async_copy(t1_hbm, t1_v, insem).start()
    pltpu.make_async_copy(val_hbm.at[pl.ds(base, _RPW)], val_v, insem).start()

    def zrow(i, _):
        def zcol(c, _2):
            zbuf[i, pl.ds(c * jnp.int32(16), 16)] = jnp.zeros((16,), jnp.float32)
            return jnp.int32(0)
        lax.fori_loop(jnp.int32(0), jnp.int32(8), zcol, jnp.int32(0))
        return jnp.int32(0)
    lax.fori_loop(jnp.int32(0), jnp.int32(_ZROWS * _BLK), zrow, jnp.int32(0))

    def fire(i, _):
        pltpu.make_async_copy(
            zbuf,
            o_hbm.at[pl.ds(bbase + i * jnp.int32(_ZROWS * _BLK), _ZROWS * _BLK)],
            zsem,
        ).start()
        return jnp.int32(0)
    lax.fori_loop(jnp.int32(0), jnp.int32(_NZ), fire, jnp.int32(0))

    pltpu.make_async_copy(x_hbm, x_v, insem).wait()
    pltpu.make_async_copy(t0_hbm, t0_v, insem).wait()
    pltpu.make_async_copy(t1_hbm, t1_v, insem).wait()
    pltpu.make_async_copy(val_hbm.at[pl.ds(base, _RPW)], val_v, insem).wait()

    def zsrow(i, _):
        def zscol(c, _2):
            srcb[i, pl.ds(c * jnp.int32(16), 16)] = jnp.zeros((16,), jnp.float32)
            return jnp.int32(0)
        lax.fori_loop(jnp.int32(0), jnp.int32(8), zscol, jnp.int32(0))
        return jnp.int32(0)
    lax.fori_loop(jnp.int32(0), jnp.int32(128), zsrow, jnp.int32(0))

    lanes = lax.broadcasted_iota(jnp.int32, (16,), 0)

    def hash_vec(v, _):
        g = v // jnp.int32(8)
        c = lax.rem(v, jnp.int32(8))
        r = base + v * jnp.int32(16) + lanes
        d = r >> jnp.int32(12)
        b = r & jnp.int32(4095)
        k = b * jnp.int32(_DEP) + d
        f = k >> jnp.int32(12)
        j = k & jnp.int32(4095)
        xg = plsc.load_gather(x_v, [j])
        x1 = xg >> jnp.int32(10)
        x0 = xg & jnp.int32(1023)
        t1 = plsc.load_gather(t1_v, [f * jnp.int32(1024) + x1])
        t0 = plsc.load_gather(t0_v, [f * jnp.int32(1024) + x0])
        s = t1.astype(jnp.uint32) + t0.astype(jnp.uint32)
        pr = jnp.uint32(_PRIME)
        rm = jnp.where(s >= pr, s - pr, s).astype(jnp.int32)
        slot = lax.rem(rm, jnp.int32(_SLOTS))
        blkb[g, pl.ds(c * jnp.int32(16), 16)] = r * jnp.int32(_BLK) + (slot >> jnp.int32(7))
        laneb[g, pl.ds(c * jnp.int32(16), 16)] = slot & jnp.int32(127)
        return jnp.int32(0)

    lax.fori_loop(jnp.int32(0), jnp.int32(_RPW // 16), hash_vec, jnp.int32(0))

    def drain(i, _):
        pltpu.make_async_copy(
            zbuf,
            o_hbm.at[pl.ds(bbase + i * jnp.int32(_ZROWS * _BLK), _ZROWS * _BLK)],
            zsem,
        ).wait()
        return jnp.int32(0)
    lax.fori_loop(jnp.int32(0), jnp.int32(_NZ), drain, jnp.int32(0))

    for g in range(_NS):
        gi = jnp.int32(g)
        for c in range(8):
            ci = jnp.int32(c)
            lane16 = laneb[gi, pl.ds(ci * jnp.int32(16), 16)]
            val16 = val_v[pl.ds(gi * jnp.int32(128) + ci * jnp.int32(16), 16)]
            pos16 = ci * jnp.int32(16) + lanes
            plsc.store_scatter(srcb, [pos16, lane16], val16)
        pltpu.sync_copy(srcb, o_hbm.at[blkb.at[gi]])
        for c in range(8):
            ci = jnp.int32(c)
            lane16 = laneb[gi, pl.ds(ci * jnp.int32(16), 16)]
            pos16 = ci * jnp.int32(16) + lanes
            plsc.store_scatter(srcb, [pos16, lane16], jnp.zeros((16,), jnp.float32))


def kernel(input_tensor, values):
    x = input_tensor.astype(jnp.int32)
    t0 = jnp.asarray(_T0)
    t1 = jnp.asarray(_T1)
    val_row = values.astype(jnp.float32).reshape(_BATCH, _DEP).T.reshape(-1)

    mesh = plsc.VectorSubcoreMesh(core_axis_name="c", subcore_axis_name="s")
    cp = pltpu.CompilerParams()
    if "needs_layout_passes" in pltpu.CompilerParams.__dataclass_fields__:
        cp = dataclasses.replace(cp, needs_layout_passes=False)
    run = pl.kernel(
        _sc_body,
        compiler_params=cp,
        out_type=jax.ShapeDtypeStruct((_NBLK, 128), jnp.float32),
        mesh=mesh,
        scratch_types=[
            pltpu.VMEM((_ZROWS * _BLK, 128), jnp.float32),
            pltpu.VMEM((_BATCH,), jnp.int32),
            pltpu.VMEM((_DEP * 1024,), jnp.int32),
            pltpu.VMEM((_DEP * 1024,), jnp.int32),
            pltpu.VMEM((_RPW,), jnp.float32),
            pltpu.VMEM((128, 128), jnp.float32),
            pltpu.VMEM((_NS, 128), jnp.int32),
            pltpu.VMEM((_NS, 128), jnp.int32),
            pltpu.SemaphoreType.DMA,
            pltpu.SemaphoreType.DMA,
        ],
    )
    out = run(x, t0, t1, val_row)
    return out.reshape(_DEP, _BATCH, _SLOTS)


# speedup vs baseline: 3.1697x; 3.1666x over previous
import dataclasses
import functools

import numpy as np
import jax
import jax.numpy as jnp
from jax import lax
from jax.experimental import pallas as pl
from jax.experimental.pallas import tpu as pltpu
from jax.experimental.pallas import tpu_sc as plsc

_HASH_SEED = 1
_DEP = 5
_SLOTS = 5120
_PRIME = 2147483647
_BATCH = 4096
_ROWS = _DEP * _BATCH

_NW = 32
_RPW = _ROWS // _NW
_CR = 8
_NCH = _RPW // _CR
_NB = 2


def _hash_tables():
    rng = np.random.RandomState(_HASH_SEED)
    A = rng.randint(1, _PRIME, size=(_DEP,)).astype(np.int64)
    B = rng.randint(0, _PRIME, size=(_DEP,)).astype(np.int64)
    v = np.arange(1024, dtype=np.int64)
    T0 = (A[:, None] * v[None, :] + B[:, None]) % _PRIME
    T1 = (A[:, None] * 1024 * v[None, :]) % _PRIME
    return (T0.reshape(-1).astype(np.int32), T1.reshape(-1).astype(np.int32))


_T0, _T1 = _hash_tables()


def _sc_body(x_hbm, t0_hbm, t1_hbm, val_hbm, o_hbm,
             zb, x_v, t0_v, t1_v, val_v, slotb, sems, insem):
    wid = (lax.axis_index("s").astype(jnp.int32) * jnp.int32(2)
           + lax.axis_index("c").astype(jnp.int32))
    base = wid * jnp.int32(_RPW)

    pltpu.make_async_copy(x_hbm, x_v, insem).start()
    pltpu.make_async_copy(t0_hbm, t0_v, insem).start()
    pltpu.make_async_copy(t1_hbm, t1_v, insem).start()
    pltpu.make_async_copy(val_hbm.at[pl.ds(base, _RPW)],
                          val_v.at[pl.ds(jnp.int32(0), _RPW)], insem).start()

    def zrow(i, _):
        def zcol(c, _2):
            zb[i // jnp.int32(_CR), lax.rem(i, jnp.int32(_CR)),
               pl.ds(c * jnp.int32(16), 16)] = jnp.zeros((16,), jnp.float32)
            return jnp.int32(0)
        lax.fori_loop(jnp.int32(0), jnp.int32(_SLOTS // 16), zcol, jnp.int32(0))
        return jnp.int32(0)
    lax.fori_loop(jnp.int32(0), jnp.int32(_NB * _CR), zrow, jnp.int32(0))

    pltpu.make_async_copy(x_hbm, x_v, insem).wait()
    pltpu.make_async_copy(t0_hbm, t0_v, insem).wait()
    pltpu.make_async_copy(t1_hbm, t1_v, insem).wait()
    pltpu.make_async_copy(val_hbm.at[pl.ds(base, _RPW)],
                          val_v.at[pl.ds(jnp.int32(0), _RPW)], insem).wait()

    lanes = lax.broadcasted_iota(jnp.int32, (16,), 0)
    mask = lanes < jnp.int32(_CR)

    def chunk(i, _):
        bi = lax.rem(i, jnp.int32(_NB))
        row0 = base + i * jnp.int32(_CR)

        r = jnp.minimum(row0 + lanes, jnp.int32(_ROWS - 1))
        d = r >> jnp.int32(12)
        b = r & jnp.int32(4095)
        k = b * jnp.int32(_DEP) + d
        f = k >> jnp.int32(12)
        j = k & jnp.int32(4095)
        xg = plsc.load_gather(x_v, [j])
        x1 = xg >> jnp.int32(10)
        x0 = xg & jnp.int32(1023)
        t1 = plsc.load_gather(t1_v, [f * jnp.int32(1024) + x1])
        t0 = plsc.load_gather(t0_v, [f * jnp.int32(1024) + x0])
        s = t1.astype(jnp.uint32) + t0.astype(jnp.uint32)
        pr = jnp.uint32(_PRIME)
        rm = jnp.where(s >= pr, s - pr, s).astype(jnp.int32)
        slot = lax.rem(rm, jnp.int32(_SLOTS))
        val16 = val_v[pl.ds(i * jnp.int32(_CR), 16)]

        @pl.when(i >= jnp.int32(_NB))
        def _recycle():
            pltpu.make_async_copy(
                zb.at[bi],
                o_hbm.at[pl.ds(row0 - jnp.int32(_NB * _CR), _CR)],
                sems.at[bi],
            ).wait()
            prev = slotb[bi, :]
            plsc.store_scatter(zb.at[bi], [lanes, prev],
                               jnp.zeros((16,), jnp.float32), mask=mask)

        plsc.store_scatter(zb.at[bi], [lanes, slot], val16, mask=mask)
        slotb[bi, :] = slot

        pltpu.make_async_copy(
            zb.at[bi],
            o_hbm.at[pl.ds(row0, _CR)],
            sems.at[bi],
        ).start()
        return jnp.int32(0)

    lax.fori_loop(jnp.int32(0), jnp.int32(_NCH), chunk, jnp.int32(0))

    def drain(i, _):
        bi = lax.rem(i, jnp.int32(_NB))
        pltpu.make_async_copy(
            zb.at[bi],
            o_hbm.at[pl.ds(base + i * jnp.int32(_CR), _CR)],
            sems.at[bi],
        ).wait()
        return jnp.int32(0)
    lax.fori_loop(jnp.int32(_NCH - _NB), jnp.int32(_NCH), drain, jnp.int32(0))


def kernel(input_tensor, values):
    x = input_tensor.astype(jnp.int32)
    t0 = jnp.asarray(_T0)
    t1 = jnp.asarray(_T1)
    val_row = values.astype(jnp.float32).reshape(_BATCH, _DEP).T.reshape(-1)

    mesh = plsc.VectorSubcoreMesh(core_axis_name="c", subcore_axis_name="s")
    cp = pltpu.CompilerParams()
    if "needs_layout_passes" in pltpu.CompilerParams.__dataclass_fields__:
        cp = dataclasses.replace(cp, needs_layout_passes=False)
    run = pl.kernel(
        _sc_body,
        compiler_params=cp,
        out_type=jax.ShapeDtypeStruct((_ROWS, _SLOTS), jnp.float32),
        mesh=mesh,
        scratch_types=[
            pltpu.VMEM((_NB, _CR, _SLOTS), jnp.float32),
            pltpu.VMEM((_BATCH,), jnp.int32),
            pltpu.VMEM((_DEP * 1024,), jnp.int32),
            pltpu.VMEM((_DEP * 1024,), jnp.int32),
            pltpu.VMEM((_RPW + 16,), jnp.float32),
            pltpu.VMEM((_NB, 16), jnp.int32),
            pltpu.SemaphoreType.DMA((_NB,)),
            pltpu.SemaphoreType.DMA,
        ],
    )
    out = run(x, t0, t1, val_row)
    return out.reshape(_DEP, _BATCH, _SLOTS)


# speedup vs baseline: 3.3293x; 1.0503x over previous
import dataclasses
import functools

import numpy as np
import jax
import jax.numpy as jnp
from jax import lax
from jax.experimental import pallas as pl
from jax.experimental.pallas import tpu as pltpu
from jax.experimental.pallas import tpu_sc as plsc

_HASH_SEED = 1
_DEP = 5
_SLOTS = 5120
_PRIME = 2147483647
_BATCH = 4096
_ROWS = _DEP * _BATCH

_NW = 32
_RPW = _ROWS // _NW
_CR = 8
_NCH = _RPW // _CR
_NB = 2


def _hash_tables():
    rng = np.random.RandomState(_HASH_SEED)
    A = rng.randint(1, _PRIME, size=(_DEP,)).astype(np.int64)
    B = rng.randint(0, _PRIME, size=(_DEP,)).astype(np.int64)
    v = np.arange(1024, dtype=np.int64)
    T0 = (A[:, None] * v[None, :] + B[:, None]) % _PRIME
    T1 = (A[:, None] * 1024 * v[None, :]) % _PRIME
    return (T0.reshape(-1).astype(np.int32), T1.reshape(-1).astype(np.int32))


_T0, _T1 = _hash_tables()


def _sc_body(x_hbm, t0_hbm, t1_hbm, val_hbm, z_hbm, o_hbm,
             zb, x_v, t0_v, t1_v, val_v, slotb, sems, insem):
    wid = (lax.axis_index("s").astype(jnp.int32) * jnp.int32(2)
           + lax.axis_index("c").astype(jnp.int32))
    base = wid * jnp.int32(_RPW)

    pltpu.make_async_copy(x_hbm, x_v, insem).start()
    pltpu.make_async_copy(t0_hbm, t0_v, insem).start()
    pltpu.make_async_copy(t1_hbm, t1_v, insem).start()
    pltpu.make_async_copy(val_hbm.at[pl.ds(base, _RPW)],
                          val_v.at[pl.ds(jnp.int32(0), _RPW)], insem).start()

    pltpu.make_async_copy(z_hbm, zb, insem).start()

    pltpu.make_async_copy(x_hbm, x_v, insem).wait()
    pltpu.make_async_copy(t0_hbm, t0_v, insem).wait()
    pltpu.make_async_copy(t1_hbm, t1_v, insem).wait()
    pltpu.make_async_copy(val_hbm.at[pl.ds(base, _RPW)],
                          val_v.at[pl.ds(jnp.int32(0), _RPW)], insem).wait()
    pltpu.make_async_copy(z_hbm, zb, insem).wait()

    lanes = lax.broadcasted_iota(jnp.int32, (16,), 0)
    mask = lanes < jnp.int32(_CR)

    def chunk(i, _):
        bi = lax.rem(i, jnp.int32(_NB))
        row0 = base + i * jnp.int32(_CR)

        r = jnp.minimum(row0 + lanes, jnp.int32(_ROWS - 1))
        d = r >> jnp.int32(12)
        b = r & jnp.int32(4095)
        k = b * jnp.int32(_DEP) + d
        f = k >> jnp.int32(12)
        j = k & jnp.int32(4095)
        xg = plsc.load_gather(x_v, [j])
        x1 = xg >> jnp.int32(10)
        x0 = xg & jnp.int32(1023)
        t1 = plsc.load_gather(t1_v, [f * jnp.int32(1024) + x1])
        t0 = plsc.load_gather(t0_v, [f * jnp.int32(1024) + x0])
        s = t1.astype(jnp.uint32) + t0.astype(jnp.uint32)
        pr = jnp.uint32(_PRIME)
        rm = jnp.where(s >= pr, s - pr, s).astype(jnp.int32)
        slot = lax.rem(rm, jnp.int32(_SLOTS))
        val16 = val_v[pl.ds(i * jnp.int32(_CR), 16)]

        @pl.when(i >= jnp.int32(_NB))
        def _recycle():
            pltpu.make_async_copy(
                zb.at[bi],
                o_hbm.at[pl.ds(row0 - jnp.int32(_NB * _CR), _CR)],
                sems.at[bi],
            ).wait()
            prev = slotb[bi, :]
            plsc.store_scatter(zb.at[bi], [lanes, prev],
                               jnp.zeros((16,), jnp.float32), mask=mask)

        plsc.store_scatter(zb.at[bi], [lanes, slot], val16, mask=mask)
        slotb[bi, :] = slot

        pltpu.make_async_copy(
            zb.at[bi],
            o_hbm.at[pl.ds(row0, _CR)],
            sems.at[bi],
        ).start()
        return jnp.int32(0)

    lax.fori_loop(jnp.int32(0), jnp.int32(_NCH), chunk, jnp.int32(0))

    def drain(i, _):
        bi = lax.rem(i, jnp.int32(_NB))
        pltpu.make_async_copy(
            zb.at[bi],
            o_hbm.at[pl.ds(base + i * jnp.int32(_CR), _CR)],
            sems.at[bi],
        ).wait()
        return jnp.int32(0)
    lax.fori_loop(jnp.int32(_NCH - _NB), jnp.int32(_NCH), drain, jnp.int32(0))


def kernel(input_tensor, values):
    x = input_tensor.astype(jnp.int32)
    t0 = jnp.asarray(_T0)
    t1 = jnp.asarray(_T1)
    val_row = values.astype(jnp.float32).reshape(_BATCH, _DEP).T.reshape(-1)

    mesh = plsc.VectorSubcoreMesh(core_axis_name="c", subcore_axis_name="s")
    cp = pltpu.CompilerParams()
    if "needs_layout_passes" in pltpu.CompilerParams.__dataclass_fields__:
        cp = dataclasses.replace(cp, needs_layout_passes=False)
    run = pl.kernel(
        _sc_body,
        compiler_params=cp,
        out_type=jax.ShapeDtypeStruct((_ROWS, _SLOTS), jnp.float32),
        mesh=mesh,
        scratch_types=[
            pltpu.VMEM((_NB, _CR, _SLOTS), jnp.float32),
            pltpu.VMEM((_BATCH,), jnp.int32),
            pltpu.VMEM((_DEP * 1024,), jnp.int32),
            pltpu.VMEM((_DEP * 1024,), jnp.int32),
            pltpu.VMEM((_RPW + 16,), jnp.float32),
            pltpu.VMEM((_NB, 16), jnp.int32),
            pltpu.SemaphoreType.DMA((_NB,)),
            pltpu.SemaphoreType.DMA,
        ],
    )
    zinit = jnp.zeros((_NB, _CR, _SLOTS), jnp.float32)
    out = run(x, t0, t1, val_row, zinit)
    return out.reshape(_DEP, _BATCH, _SLOTS)
